# baseline (speedup 1.0000x reference)
import functools

import jax
import jax.numpy as jnp
from jax import lax
from jax.experimental import pallas as pl
from jax.experimental.pallas import tpu as pltpu
from jax.experimental.pallas import tpu_sc as plsc

VOCAB = 1000000
D_MODEL = 64
ROWS = 4096 * 200
K = 128
NUM_CHUNKS = ROWS // K


def _build_gather(num_workers):
    chunks_per_w = NUM_CHUNKS // num_workers
    mesh = plsc.VectorSubcoreMesh(core_axis_name="c", subcore_axis_name="s")
    nc = 2

    @functools.partial(
        pl.kernel,
        mesh=mesh,
        out_type=jax.ShapeDtypeStruct((ROWS, D_MODEL), jnp.float32),
        compiler_params=pltpu.CompilerParams(use_tc_tiling_on_sc=False),
        scratch_types=[
            pltpu.VMEM((chunks_per_w, K), jnp.int32),
            pltpu.VMEM((K, D_MODEL), jnp.float32),
            pltpu.SemaphoreType.DMA,
        ],
    )
    def gather_kernel(idx_hbm, table_hbm, out_hbm, idx_v, rows_v, sem):
        wid = lax.axis_index("s") * nc + lax.axis_index("c")
        cbase = wid * chunks_per_w
        pltpu.sync_copy(idx_hbm.at[pl.ds(cbase, chunks_per_w)], idx_v)

        def body(j, carry):
            pltpu.async_copy(table_hbm.at[idx_v.at[j]], rows_v, sem).wait()
            pltpu.sync_copy(rows_v, out_hbm.at[pl.ds((cbase + j) * K, K)])
            return carry

        lax.fori_loop(0, chunks_per_w, body, 0, unroll=False)

    return gather_kernel


def kernel(inputs, table):
    info = plsc.get_sparse_core_info()
    num_workers = info.num_cores * info.num_subcores
    idx = inputs.reshape(NUM_CHUNKS, K).astype(jnp.int32)
    out = _build_gather(num_workers)(idx, table)
    return out.reshape(inputs.shape + (D_MODEL,))


# speedup vs baseline: 1.1152x; 1.1152x over previous
import functools

import jax
import jax.numpy as jnp
from jax import lax
from jax.experimental import pallas as pl
from jax.experimental.pallas import tpu as pltpu
from jax.experimental.pallas import tpu_sc as plsc

VOCAB = 1000000
D_MODEL = 64
ROWS = 4096 * 200
K = 128
NUM_CHUNKS = ROWS // K
NBUF = 8


def _build_gather(num_workers):
    chunks_per_w = NUM_CHUNKS // num_workers
    mesh = plsc.VectorSubcoreMesh(core_axis_name="c", subcore_axis_name="s")
    nc = 2

    @functools.partial(
        pl.kernel,
        mesh=mesh,
        out_type=jax.ShapeDtypeStruct((ROWS, D_MODEL), jnp.float32),
        compiler_params=pltpu.CompilerParams(use_tc_tiling_on_sc=False),
        scratch_types=[
            pltpu.VMEM((chunks_per_w, K), jnp.int32),
            pltpu.VMEM((NBUF, K, D_MODEL), jnp.float32),
            pltpu.SemaphoreType.DMA((NBUF,)),
            pltpu.SemaphoreType.DMA((NBUF,)),
        ],
    )
    def gather_kernel(idx_hbm, table_hbm, out_hbm, idx_v, rows_v, gsem, wsem):
        wid = lax.axis_index("s") * nc + lax.axis_index("c")
        cbase = wid * chunks_per_w
        pltpu.sync_copy(idx_hbm.at[pl.ds(cbase, chunks_per_w)], idx_v)

        for b in range(NBUF):
            pltpu.async_copy(table_hbm.at[idx_v.at[b]], rows_v.at[b],
                             gsem.at[b])

        def outer(i, carry):
            g0 = i * NBUF
            for b in range(NBUF):
                g = g0 + b
                pltpu.make_async_copy(table_hbm.at[idx_v.at[b]],
                                      rows_v.at[b], gsem.at[b]).wait()
                dst = out_hbm.at[pl.ds((cbase + g) * K, K)]
                pltpu.async_copy(rows_v.at[b], dst, wsem.at[b])

                @pl.when(g + NBUF < chunks_per_w)
                def _():
                    pltpu.make_async_copy(rows_v.at[b], dst,
                                          wsem.at[b]).wait()
                    pltpu.async_copy(
                        table_hbm.at[idx_v.at[g + NBUF]],
                        rows_v.at[b], gsem.at[b])
            return carry

        lax.fori_loop(0, chunks_per_w // NBUF, outer, 0, unroll=False)

        for b in range(NBUF):
            pltpu.make_async_copy(
                rows_v.at[b], out_hbm.at[pl.ds(cbase * K, K)],
                wsem.at[b]).wait()

    return gather_kernel


def kernel(inputs, table):
    info = plsc.get_sparse_core_info()
    num_workers = info.num_cores * info.num_subcores
    idx = inputs.reshape(NUM_CHUNKS, K).astype(jnp.int32)
    out = _build_gather(num_workers)(idx, table)
    return out.reshape(inputs.shape + (D_MODEL,))


# speedup vs baseline: 1.1180x; 1.0025x over previous
import functools

import jax
import jax.numpy as jnp
from jax import lax
from jax.experimental import pallas as pl
from jax.experimental.pallas import tpu as pltpu
from jax.experimental.pallas import tpu_sc as plsc

VOCAB = 1000000
D_MODEL = 64
ROWS = 4096 * 200
K = 128
NUM_CHUNKS = ROWS // K
PAIRS = NUM_CHUNKS // 2
NBUF = 4


def _build_gather(num_workers):
    chunks_per_w = NUM_CHUNKS // num_workers
    pairs_per_w = chunks_per_w // 2
    mesh = plsc.VectorSubcoreMesh(core_axis_name="c", subcore_axis_name="s")
    nc = 2

    @functools.partial(
        pl.kernel,
        mesh=mesh,
        out_type=jax.ShapeDtypeStruct((ROWS, D_MODEL), jnp.float32),
        compiler_params=pltpu.CompilerParams(use_tc_tiling_on_sc=False),
        scratch_types=[
            pltpu.VMEM((chunks_per_w, K), jnp.int32),
            pltpu.VMEM((NBUF, 2 * K, D_MODEL), jnp.float32),
            pltpu.SemaphoreType.DMA((NBUF,)),
            pltpu.SemaphoreType.DMA((NBUF,)),
        ],
    )
    def gather_kernel(idx_hbm, table_hbm, out_hbm, idx_v, rows_v, gsem, wsem):
        wid = lax.axis_index("s") * nc + lax.axis_index("c")
        cbase = wid * chunks_per_w
        pltpu.sync_copy(idx_hbm.at[pl.ds(cbase, chunks_per_w)], idx_v)

        def fire_pair(p, b):
            pltpu.async_copy(table_hbm.at[idx_v.at[2 * p]],
                             rows_v.at[b, pl.ds(0, K)], gsem.at[b])
            pltpu.async_copy(table_hbm.at[idx_v.at[2 * p + 1]],
                             rows_v.at[b, pl.ds(K, K)], gsem.at[b])

        for b in range(NBUF):
            fire_pair(b, b)

        def outer(i, carry):
            p0 = i * NBUF
            for b in range(NBUF):
                p = p0 + b
                pltpu.make_async_copy(out_hbm.at[pl.ds(0, 2 * K)],
                                      rows_v.at[b], gsem.at[b]).wait()
                dst = out_hbm.at[pl.ds((cbase + 2 * p) * K, 2 * K)]
                pltpu.async_copy(rows_v.at[b], dst, wsem.at[b])

                @pl.when(p + NBUF < pairs_per_w)
                def _():
                    pltpu.make_async_copy(rows_v.at[b], dst,
                                          wsem.at[b]).wait()
                    fire_pair(p + NBUF, b)
            return carry

        lax.fori_loop(0, pairs_per_w // NBUF, outer, 0, unroll=False)

        for b in range(NBUF):
            pltpu.make_async_copy(
                rows_v.at[b], out_hbm.at[pl.ds(cbase * K, 2 * K)],
                wsem.at[b]).wait()

    return gather_kernel


def kernel(inputs, table):
    info = plsc.get_sparse_core_info()
    num_workers = info.num_cores * info.num_subcores
    idx = inputs.reshape(NUM_CHUNKS, K).astype(jnp.int32)
    out = _build_gather(num_workers)(idx, table)
    return out.reshape(inputs.shape + (D_MODEL,))
